# baseline (speedup 1.0000x reference)
import dataclasses
import functools

import jax
import jax.numpy as jnp
from jax import lax
from jax.experimental import pallas as pl
from jax.experimental.pallas import tpu as pltpu
from jax.experimental.pallas import tpu_sc as plsc

N = 10000
E = 320000
D = 128
D2 = D // 2
ALPHA = 0.1

NC = 2
NS = 16
NW = NC * NS

EPW = E // NW
DCH = 64
DNCH = 160
EPWP = DNCH * DCH

CH = 128
NCHT = 80
WCH = 16
NWIN = NCHT // WCH

N2 = 10240
NA = 10240
RPT = NA // NS
BLK = 1024
NG = 10

_mesh = plsc.VectorSubcoreMesh(core_axis_name="c", subcore_axis_name="s")

_sc_params = pltpu.CompilerParams()
if "needs_layout_passes" in pltpu.CompilerParams.__dataclass_fields__:
    _sc_params = dataclasses.replace(_sc_params, needs_layout_passes=False)
if "use_tc_tiling_on_sc" in pltpu.CompilerParams.__dataclass_fields__:
    _sc_params = dataclasses.replace(_sc_params, use_tc_tiling_on_sc=False)


@functools.partial(
    pl.kernel,
    out_type=jax.ShapeDtypeStruct((NW, N2), jnp.float32),
    mesh=_mesh,
    scratch_types=[
        pltpu.VMEM((N2,), jnp.float32),
        pltpu.VMEM((DNCH, DCH), jnp.int32),
    ],
    compiler_params=_sc_params,
)
def _sc_degree_kernel(dst_hbm, out_hbm, hist, idxs):
    c = lax.axis_index("c")
    s = lax.axis_index("s")
    wid = c * NS + s

    @pl.loop(0, N2, step=16)
    def _(i):
        hist[pl.ds(i, 16)] = jnp.zeros((16,), jnp.float32)

    pltpu.sync_copy(dst_hbm.at[wid], idxs)
    ones = jnp.ones((16,), jnp.float32)

    @pl.loop(0, DNCH)
    def _(j):
        @pl.loop(0, DCH, step=16)
        def _(k):
            plsc.addupdate_scatter(hist, [idxs[j, pl.ds(k, 16)]], ones)

    pltpu.sync_copy(hist, out_hbm.at[wid])


@functools.partial(
    pl.kernel,
    out_type=jax.ShapeDtypeStruct((NC, NA, D), jnp.bfloat16),
    mesh=_mesh,
    scratch_types=[
        pltpu.VMEM((2, WCH, CH), jnp.int32),
        pltpu.VMEM((2, WCH, CH), jnp.int32),
        pltpu.VMEM((CH,), jnp.int32),
        pltpu.VMEM((CH,), jnp.int32),
        pltpu.VMEM((CH,), jnp.int32),
        pltpu.VMEM((CH,), jnp.int32),
        pltpu.VMEM((CH, D), jnp.bfloat16),
        pltpu.VMEM((CH, D), jnp.bfloat16),
        pltpu.VMEM_SHARED((NA, D), jnp.bfloat16),
        pltpu.VMEM_SHARED((NA, D), jnp.bfloat16),
        pltpu.SemaphoreType.DMA,
        pltpu.SemaphoreType.DMA,
        pltpu.SemaphoreType.DMA,
        pltpu.SemaphoreType.DMA,
        pltpu.SemaphoreType.DMA,
    ],
    compiler_params=_sc_params,
)
def _sc_agg_kernel(h_hbm, src_hbm, dst_hbm, out_hbm, sidxw, didxw,
                   sg0, sg1, sd0, sd1, rows0, rows1, hsp, acc,
                   g0, g1, s0, s1, wsem):
    c = lax.axis_index("c")
    s = lax.axis_index("s")
    wid = c * NS + s

    @pl.when(s < NS - 1)
    def _():
        pltpu.sync_copy(h_hbm.at[pl.ds(s * RPT, RPT)],
                        hsp.at[pl.ds(s * RPT, RPT)])

    @pl.when(s == NS - 1)
    def _():
        pltpu.sync_copy(h_hbm.at[pl.ds((NS - 1) * RPT, N - (NS - 1) * RPT)],
                        hsp.at[pl.ds((NS - 1) * RPT, N - (NS - 1) * RPT)])

    @pl.loop(0, CH)
    def _(i):
        @pl.loop(0, D, step=32)
        def _(k):
            rows0[i, pl.ds(k, 32)] = jnp.zeros((32,), jnp.bfloat16)

    @pl.loop(0, RPT, step=CH)
    def _(r):
        pltpu.sync_copy(rows0, acc.at[pl.ds(s * RPT + r, CH)])

    pltpu.sync_copy(src_hbm.at[wid, pl.ds(0, WCH)], sidxw.at[0])
    pltpu.sync_copy(dst_hbm.at[wid, pl.ds(0, WCH)], didxw.at[0])
    plsc.subcore_barrier()

    def stage(dref, sref, p, k):
        @pl.loop(0, CH, step=16)
        def _(k2):
            dref[pl.ds(k2, 16)] = sref[p, k, pl.ds(k2, 16)]

    def gather(p, k, sg, buf, sem):
        stage(sg, sidxw, p, k)
        pltpu.async_copy(hsp.at[sg], buf, sem)

    def scat(p, k, sd, buf, sem):
        stage(sd, didxw, p, k)
        pltpu.async_copy(buf, acc.at[sd], sem, add=True)

    def gwait(buf, sem):
        pltpu.make_async_copy(hsp.at[pl.ds(0, CH)], buf, sem).wait()

    def swait(buf, sem):
        pltpu.make_async_copy(buf, acc.at[pl.ds(0, CH)], sem).wait()

    @pl.loop(0, NWIN)
    def _(w):
        p = w % 2

        @pl.when(w > 0)
        def _():
            pltpu.make_async_copy(
                src_hbm.at[wid, pl.ds(0, WCH)], sidxw.at[0], wsem).wait()
            pltpu.make_async_copy(
                dst_hbm.at[wid, pl.ds(0, WCH)], didxw.at[0], wsem).wait()

        gather(p, 0, sg0, rows0, g0)
        gather(p, 1, sg1, rows1, g1)

        @pl.when(w + 1 < NWIN)
        def _():
            pltpu.async_copy(
                src_hbm.at[wid, pl.ds((w + 1) * WCH, WCH)],
                sidxw.at[1 - p], wsem)
            pltpu.async_copy(
                dst_hbm.at[wid, pl.ds((w + 1) * WCH, WCH)],
                didxw.at[1 - p], wsem)

        @pl.loop(0, WCH - 2, step=2)
        def _(k):
            gwait(rows0, g0)
            scat(p, k, sd0, rows0, s0)
            gwait(rows1, g1)
            scat(p, k + 1, sd1, rows1, s1)
            swait(rows0, s0)
            gather(p, k + 2, sg0, rows0, g0)
            swait(rows1, s1)
            gather(p, k + 3, sg1, rows1, g1)

        kl = WCH - 2
        gwait(rows0, g0)
        scat(p, kl, sd0, rows0, s0)
        gwait(rows1, g1)
        scat(p, kl + 1, sd1, rows1, s1)
        swait(rows0, s0)
        swait(rows1, s1)

    plsc.subcore_barrier()

    @pl.loop(0, RPT, step=CH)
    def _(r):
        pltpu.sync_copy(
            acc.at[pl.ds(s * RPT + r, CH)],
            out_hbm.at[c, pl.ds(s * RPT + r, CH)],
        )


def _tc_prep_body(feat_ref, init_ref, wt_ref, hist_ref, h2_ref, base_ref):
    degs = jnp.maximum(jnp.sum(hist_ref[...], axis=0), 1.0)
    norm = lax.rsqrt(degs)[:, None]
    x = feat_ref[...]
    h = jnp.dot(x, wt_ref[...], preferred_element_type=jnp.float32,
                precision=lax.Precision.HIGHEST) * norm
    h2_ref[...] = h.astype(jnp.bfloat16)
    base_ref[...] = (1.0 - ALPHA) * x + (ALPHA / degs[:, None]) * init_ref[...]


def _tc_final_body(agg_ref, base_ref, hist_ref, out_ref):
    degs = jnp.maximum(jnp.sum(hist_ref[...], axis=0), 1.0)
    norm = lax.rsqrt(degs)[:, None]
    agg = (agg_ref[0].astype(jnp.float32) + agg_ref[1].astype(jnp.float32))
    out_ref[...] = base_ref[...] + ALPHA * agg * norm


def kernel(features, initial_features, edge_index, W):
    src = edge_index[0]
    dst = edge_index[1]

    dst_deg = jnp.concatenate(
        [dst.reshape(NW, EPW),
         jnp.full((NW, EPWP - EPW), NA - 1, jnp.int32)], axis=1
    ).reshape(NW, DNCH, DCH)

    src_agg = jnp.concatenate(
        [src.reshape(NW, EPW), jnp.zeros((NW, EPWP - EPW), jnp.int32)], axis=1
    ).reshape(NW, NCHT, CH)
    dst_agg = dst_deg.reshape(NW, NCHT, CH)

    hists = _sc_degree_kernel(dst_deg)

    h2, base = pl.pallas_call(
        _tc_prep_body,
        grid=(NG,),
        in_specs=[
            pl.BlockSpec((BLK, D), lambda i: (i, 0)),
            pl.BlockSpec((BLK, D), lambda i: (i, 0)),
            pl.BlockSpec((D, D), lambda i: (0, 0)),
            pl.BlockSpec((NW, BLK), lambda i: (0, i)),
        ],
        out_specs=[
            pl.BlockSpec((BLK, D), lambda i: (i, 0)),
            pl.BlockSpec((BLK, D), lambda i: (i, 0)),
        ],
        out_shape=[
            jax.ShapeDtypeStruct((N, D), jnp.bfloat16),
            jax.ShapeDtypeStruct((N, D), jnp.float32),
        ],
    )(features, initial_features, W.T, hists)

    aggs = _sc_agg_kernel(h2, src_agg, dst_agg)

    out = pl.pallas_call(
        _tc_final_body,
        grid=(NG,),
        in_specs=[
            pl.BlockSpec((NC, BLK, D), lambda i: (0, i, 0)),
            pl.BlockSpec((BLK, D), lambda i: (i, 0)),
            pl.BlockSpec((NW, BLK), lambda i: (0, i)),
        ],
        out_specs=pl.BlockSpec((BLK, D), lambda i: (i, 0)),
        out_shape=jax.ShapeDtypeStruct((N, D), jnp.float32),
    )(aggs, base, hists)
    return out


# speedup vs baseline: 1.0195x; 1.0195x over previous
import dataclasses
import functools

import jax
import jax.numpy as jnp
from jax import lax
from jax.experimental import pallas as pl
from jax.experimental.pallas import tpu as pltpu
from jax.experimental.pallas import tpu_sc as plsc

N = 10000
E = 320000
D = 128
D2 = D // 2
ALPHA = 0.1

NC = 2
NS = 16
NW = NC * NS

EPW = E // NW
DCH = 64
DNCH = 160
EPWP = DNCH * DCH

EPT = E // NS
CH = 64
NCHT = 320
EPTP = NCHT * CH
WCH = 32
NWIN = NCHT // WCH

N2 = 10240
NA = 10240
RPT = NA // NS
BLK = 1024
NG = 10

_mesh = plsc.VectorSubcoreMesh(core_axis_name="c", subcore_axis_name="s")

_sc_params = pltpu.CompilerParams()
if "needs_layout_passes" in pltpu.CompilerParams.__dataclass_fields__:
    _sc_params = dataclasses.replace(_sc_params, needs_layout_passes=False)
if "use_tc_tiling_on_sc" in pltpu.CompilerParams.__dataclass_fields__:
    _sc_params = dataclasses.replace(_sc_params, use_tc_tiling_on_sc=False)


@functools.partial(
    pl.kernel,
    out_type=jax.ShapeDtypeStruct((NW, N2), jnp.float32),
    mesh=_mesh,
    scratch_types=[
        pltpu.VMEM((N2,), jnp.float32),
        pltpu.VMEM((DNCH, DCH), jnp.int32),
    ],
    compiler_params=_sc_params,
)
def _sc_degree_kernel(dst_hbm, out_hbm, hist, idxs):
    c = lax.axis_index("c")
    s = lax.axis_index("s")
    wid = c * NS + s

    @pl.loop(0, N2, step=16)
    def _(i):
        hist[pl.ds(i, 16)] = jnp.zeros((16,), jnp.float32)

    pltpu.sync_copy(dst_hbm.at[wid], idxs)
    ones = jnp.ones((16,), jnp.float32)

    @pl.loop(0, DNCH)
    def _(j):
        @pl.loop(0, DCH, step=16)
        def _(k):
            plsc.addupdate_scatter(hist, [idxs[j, pl.ds(k, 16)]], ones)

    pltpu.sync_copy(hist, out_hbm.at[wid])


@functools.partial(
    pl.kernel,
    out_type=jax.ShapeDtypeStruct((NC, NA, D2), jnp.bfloat16),
    mesh=_mesh,
    scratch_types=[
        pltpu.VMEM((2, WCH, CH), jnp.int32),
        pltpu.VMEM((2, WCH, CH), jnp.int32),
        pltpu.VMEM((CH,), jnp.int32),
        pltpu.VMEM((CH,), jnp.int32),
        pltpu.VMEM((CH,), jnp.int32),
        pltpu.VMEM((CH,), jnp.int32),
        pltpu.VMEM((CH, D2), jnp.bfloat16),
        pltpu.VMEM((CH, D2), jnp.bfloat16),
        pltpu.VMEM_SHARED((NA, D2), jnp.bfloat16),
        pltpu.VMEM_SHARED((NA, D2), jnp.bfloat16),
        pltpu.SemaphoreType.DMA,
        pltpu.SemaphoreType.DMA,
        pltpu.SemaphoreType.DMA,
        pltpu.SemaphoreType.DMA,
        pltpu.SemaphoreType.DMA,
    ],
    compiler_params=_sc_params,
)
def _sc_agg_kernel(h2_hbm, src_hbm, dst_hbm, out_hbm, sidxw, didxw,
                   sg0, sg1, sd0, sd1, rows0, rows1, hsp, acc,
                   g0, g1, s0, s1, wsem):
    c = lax.axis_index("c")
    s = lax.axis_index("s")

    @pl.when(s < NS - 1)
    def _():
        pltpu.sync_copy(h2_hbm.at[c, pl.ds(s * RPT, RPT)],
                        hsp.at[pl.ds(s * RPT, RPT)])

    @pl.when(s == NS - 1)
    def _():
        pltpu.sync_copy(h2_hbm.at[c, pl.ds((NS - 1) * RPT, N - (NS - 1) * RPT)],
                        hsp.at[pl.ds((NS - 1) * RPT, N - (NS - 1) * RPT)])

    @pl.loop(0, CH)
    def _(i):
        @pl.loop(0, D2, step=32)
        def _(k):
            rows0[i, pl.ds(k, 32)] = jnp.zeros((32,), jnp.bfloat16)

    @pl.loop(0, RPT, step=CH)
    def _(r):
        pltpu.sync_copy(rows0, acc.at[pl.ds(s * RPT + r, CH)])

    pltpu.sync_copy(src_hbm.at[s, pl.ds(0, WCH)], sidxw.at[0])
    pltpu.sync_copy(dst_hbm.at[s, pl.ds(0, WCH)], didxw.at[0])
    plsc.subcore_barrier()

    def stage(dref, sref, p, k):
        @pl.loop(0, CH, step=16)
        def _(k2):
            dref[pl.ds(k2, 16)] = sref[p, k, pl.ds(k2, 16)]

    def gather(p, k, sg, buf, sem):
        stage(sg, sidxw, p, k)
        pltpu.async_copy(hsp.at[sg], buf, sem)

    def scat(p, k, sd, buf, sem):
        stage(sd, didxw, p, k)
        pltpu.async_copy(buf, acc.at[sd], sem, add=True)

    def gwait(buf, sem):
        pltpu.make_async_copy(hsp.at[pl.ds(0, CH)], buf, sem).wait()

    def swait(buf, sem):
        pltpu.make_async_copy(buf, acc.at[pl.ds(0, CH)], sem).wait()

    @pl.loop(0, NWIN)
    def _(w):
        p = w % 2

        @pl.when(w > 0)
        def _():
            pltpu.make_async_copy(
                src_hbm.at[s, pl.ds(0, WCH)], sidxw.at[0], wsem).wait()
            pltpu.make_async_copy(
                dst_hbm.at[s, pl.ds(0, WCH)], didxw.at[0], wsem).wait()

        gather(p, 0, sg0, rows0, g0)
        gather(p, 1, sg1, rows1, g1)

        @pl.when(w + 1 < NWIN)
        def _():
            pltpu.async_copy(
                src_hbm.at[s, pl.ds((w + 1) * WCH, WCH)],
                sidxw.at[1 - p], wsem)
            pltpu.async_copy(
                dst_hbm.at[s, pl.ds((w + 1) * WCH, WCH)],
                didxw.at[1 - p], wsem)

        @pl.loop(0, WCH - 2, step=2)
        def _(k):
            gwait(rows0, g0)
            scat(p, k, sd0, rows0, s0)
            gwait(rows1, g1)
            scat(p, k + 1, sd1, rows1, s1)
            swait(rows0, s0)
            gather(p, k + 2, sg0, rows0, g0)
            swait(rows1, s1)
            gather(p, k + 3, sg1, rows1, g1)

        kl = WCH - 2
        gwait(rows0, g0)
        scat(p, kl, sd0, rows0, s0)
        gwait(rows1, g1)
        scat(p, kl + 1, sd1, rows1, s1)
        swait(rows0, s0)
        swait(rows1, s1)

    plsc.subcore_barrier()

    @pl.loop(0, RPT, step=CH)
    def _(r):
        pltpu.sync_copy(
            acc.at[pl.ds(s * RPT + r, CH)],
            out_hbm.at[c, pl.ds(s * RPT + r, CH)],
        )


def _tc_prep_body(feat_ref, init_ref, wt_ref, hist_ref, h2_ref, base_ref):
    degs = jnp.maximum(jnp.sum(hist_ref[...], axis=0), 1.0)
    norm = lax.rsqrt(degs)[:, None]
    x = feat_ref[...]
    h = jnp.dot(x, wt_ref[...], preferred_element_type=jnp.float32,
                precision=lax.Precision.HIGHEST) * norm
    h2_ref[0] = h[:, :D2].astype(jnp.bfloat16)
    h2_ref[1] = h[:, D2:].astype(jnp.bfloat16)
    base_ref[...] = (1.0 - ALPHA) * x + (ALPHA / degs[:, None]) * init_ref[...]


def _tc_final_body(agg_ref, base_ref, hist_ref, out_ref):
    degs = jnp.maximum(jnp.sum(hist_ref[...], axis=0), 1.0)
    norm = lax.rsqrt(degs)[:, None]
    agg = jnp.concatenate([agg_ref[0], agg_ref[1]], axis=1).astype(jnp.float32)
    out_ref[...] = base_ref[...] + ALPHA * agg * norm


def kernel(features, initial_features, edge_index, W):
    src = edge_index[0]
    dst = edge_index[1]

    dst_deg = jnp.concatenate(
        [dst.reshape(NW, EPW),
         jnp.full((NW, EPWP - EPW), NA - 1, jnp.int32)], axis=1
    ).reshape(NW, DNCH, DCH)

    src_agg = jnp.concatenate(
        [src.reshape(NS, EPT), jnp.zeros((NS, EPTP - EPT), jnp.int32)], axis=1
    ).reshape(NS, NCHT, CH)
    dst_agg = jnp.concatenate(
        [dst.reshape(NS, EPT),
         jnp.full((NS, EPTP - EPT), NA - 1, jnp.int32)], axis=1
    ).reshape(NS, NCHT, CH)

    hists = _sc_degree_kernel(dst_deg)

    h2, base = pl.pallas_call(
        _tc_prep_body,
        grid=(NG,),
        in_specs=[
            pl.BlockSpec((BLK, D), lambda i: (i, 0)),
            pl.BlockSpec((BLK, D), lambda i: (i, 0)),
            pl.BlockSpec((D, D), lambda i: (0, 0)),
            pl.BlockSpec((NW, BLK), lambda i: (0, i)),
        ],
        out_specs=[
            pl.BlockSpec((2, BLK, D2), lambda i: (0, i, 0)),
            pl.BlockSpec((BLK, D), lambda i: (i, 0)),
        ],
        out_shape=[
            jax.ShapeDtypeStruct((2, N, D2), jnp.bfloat16),
            jax.ShapeDtypeStruct((N, D), jnp.float32),
        ],
    )(features, initial_features, W.T, hists)

    aggs = _sc_agg_kernel(h2, src_agg, dst_agg)

    out = pl.pallas_call(
        _tc_final_body,
        grid=(NG,),
        in_specs=[
            pl.BlockSpec((NC, BLK, D2), lambda i: (0, i, 0)),
            pl.BlockSpec((BLK, D), lambda i: (i, 0)),
            pl.BlockSpec((NW, BLK), lambda i: (0, i)),
        ],
        out_specs=pl.BlockSpec((BLK, D), lambda i: (i, 0)),
        out_shape=jax.ShapeDtypeStruct((N, D), jnp.float32),
    )(aggs, base, hists)
    return out
